# baseline (speedup 1.0000x reference)
import jax
import jax.numpy as jnp
from jax.experimental import pallas as pl
from jax.experimental.pallas import tpu as pltpu

_B, _N2, _N1, _F, _A, _O, _L = 1024, 10, 25, 256, 128, 256, 50
_N1P, _N2P = 32, 16
_BB = 32
_STEPS = _B // _BB
_CB = 64


def _l2n(x):
    return x * jax.lax.rsqrt(jnp.maximum(jnp.sum(x * x, axis=-1, keepdims=True), 1e-12))


def _start_x2_copies(x2_hbm, buf_ref, sems, step, slot):
    for b in range(_BB):
        for n in range(_N2):
            pltpu.make_async_copy(
                x2_hbm.at[pl.ds(step * _BB + b, 1), pl.ds(n, 1)],
                buf_ref.at[slot, pl.ds(b, 1), pl.ds(n, 1)],
                sems.at[slot],
            ).start()


def _wait_x2_copies(x2_hbm, buf_ref, sems, step, slot):
    pltpu.make_async_copy(
        x2_hbm.at[pl.ds(step * _BB, _BB)],
        buf_ref.at[slot],
        sems.at[slot],
    ).wait()


def _fused(x0_ref, x1_ref, x2_hbm, wagg0_ref, w0s_ref, w0a_ref, wagg1_ref,
           w1s_ref, w1a_ref, wcls_ref, out_ref, buf_ref, agg0_ref, sems):
    i = pl.program_id(0)
    slot = jax.lax.rem(i, 2)

    @pl.when(i == 0)
    def _prologue():
        _start_x2_copies(x2_hbm, buf_ref, sems, 0, 0)

    @pl.when(i + 1 < _STEPS)
    def _prefetch():
        _start_x2_copies(x2_hbm, buf_ref, sems, i + 1, jax.lax.rem(i + 1, 2))

    _wait_x2_copies(x2_hbm, buf_ref, sems, i, slot)

    x2 = jnp.pad(buf_ref[slot], ((0, 0), (0, 0), (0, _N1P - _N1), (0, 0)))
    x2 = x2.reshape(_BB * _N2 * _N1P, _F).astype(jnp.bfloat16)
    t = jnp.maximum(jnp.dot(x2, wagg0_ref[...], preferred_element_type=jnp.float32), 0.0)
    s0 = jnp.sum(t.reshape(_BB, _N2, _N1P, _A), axis=2)
    s0 = jnp.pad(s0, ((0, 0), (0, _N2P - _N2), (0, 0))).reshape(_BB * _N2P, _A)
    agg0_ref[pl.ds(i * _BB * _N2P, _BB * _N2P), :] = (
        (s0 * (1.0 / _N1)).astype(jnp.bfloat16))

    @pl.when(i % 2 == 1)
    def _tail():
        k = i // 2
        x1 = jnp.pad(x1_ref[...], ((0, 0), (0, _N2P - _N2), (0, 0)))
        x1 = x1.reshape(_CB * _N2P, _F).astype(jnp.bfloat16)
        a0 = agg0_ref[pl.ds(k * _CB * _N2P, _CB * _N2P), :]
        h1 = jnp.maximum(
            jnp.dot(x1, w0s_ref[...], preferred_element_type=jnp.float32)
            + jnp.dot(a0, w0a_ref[...], preferred_element_type=jnp.float32), 0.0)
        h1 = _l2n(h1).astype(jnp.bfloat16)
        g = jnp.maximum(jnp.dot(h1, wagg1_ref[...], preferred_element_type=jnp.float32), 0.0)
        agg1 = jnp.sum(g.reshape(_CB, _N2P, _A), axis=1) * (1.0 / _N2)
        h0 = (jnp.dot(x0_ref[...], w1s_ref[...], preferred_element_type=jnp.float32)
              + jnp.dot(agg1, w1a_ref[...], preferred_element_type=jnp.float32))
        h0 = _l2n(_l2n(h0))
        out_ref[...] = jnp.maximum(
            jnp.dot(h0, wcls_ref[...], preferred_element_type=jnp.float32), 0.0)


def _full(shape):
    return pl.BlockSpec(shape, lambda i: (0,) * len(shape))


def kernel(x0, x1, x2, Wagg0, Wagg1, Wcomb0, Wcomb1, Wcls):
    w0s = Wcomb0[:_F].astype(jnp.bfloat16)
    w0a = Wcomb0[_F:].astype(jnp.bfloat16)
    w1s, w1a = Wcomb1[:_F], Wcomb1[_F:]
    return pl.pallas_call(
        _fused,
        grid=(_STEPS,),
        in_specs=[
            pl.BlockSpec((_CB, _F), lambda i: (i // 2, 0)),
            pl.BlockSpec((_CB, _N2, _F), lambda i: (i // 2, 0, 0)),
            pl.BlockSpec(memory_space=pl.ANY),
            _full((_F, _A)), _full((_F, _O)), _full((_A, _O)),
            _full((_O, _A)), _full((_F, _O)), _full((_A, _O)),
            _full((_O, _L)),
        ],
        out_specs=pl.BlockSpec((_CB, _L), lambda i: (i // 2, 0)),
        out_shape=jax.ShapeDtypeStruct((_B, _L), jnp.float32),
        scratch_shapes=[
            pltpu.VMEM((2, _BB, _N2, _N1, _F), jnp.float32),
            pltpu.VMEM((_B * _N2P, _A), jnp.bfloat16),
            pltpu.SemaphoreType.DMA((2,)),
        ],
        compiler_params=pltpu.CompilerParams(dimension_semantics=("arbitrary",)),
    )(x0, x1, x2, Wagg0.astype(jnp.bfloat16), w0s, w0a,
      Wagg1.astype(jnp.bfloat16), w1s, w1a, Wcls)
